# baseline (speedup 1.0000x reference)
import jax
import jax.numpy as jnp
from jax import lax
from jax.experimental import pallas as pl
from jax.experimental.pallas import tpu as pltpu
from jax.experimental.pallas import tpu_sc as plsc

N = 1024
T = 200
D_OBS = 128
N_ACT = 16
B = 1024
S = 50
VR = T - S + 1

NW = 32
BPW = B // NW
NCH = BPW // 2
CW = 112
VRP = 152

_i32 = jnp.int32


def _bcast(v, ref):
    return plsc.load_gather(ref, [jnp.full((16,), v, _i32)])


def _search(load_mid, t, hi, steps):
    lo0 = jnp.zeros((16,), _i32)
    hi0 = jnp.full((16,), hi, _i32)

    def step(_, carry):
        lo, hi = carry
        mid = lax.shift_right_arithmetic(lo + hi, jnp.full((16,), 1, _i32))
        pred = load_mid(mid) <= t
        return (jnp.where(pred, mid + 1, lo), jnp.where(pred, hi, mid))

    lo, _ = lax.fori_loop(0, steps, step, (lo0, hi0))
    return lo


def _body(cdf_h, tcdf_h, u1_h, u2_h, off_h,
          obs2d, ap2d, act_h, rew_h, don_h, val_h, ret_h, wgt_h,
          obs_o, ap_o, act_o, rew_o, don_o, val_o, ret_o, wgt_o,
          cdf_v, tcdf_v, u1_v, u2_v, ti_v, off_v, st_v, tib_v, idx_v,
          rows_v, sout_v, obs_b0, obs_b1, ap_b,
          sem_t, sem_row, sem_ap, sem_o0, sem_o1):
    wid = lax.axis_index("s") * 2 + lax.axis_index("c")
    base = wid * BPW
    base50 = base * S

    pltpu.sync_copy(cdf_h, cdf_v)
    pltpu.sync_copy(u1_h.at[pl.ds(base, BPW)], u1_v)
    pltpu.sync_copy(u2_h.at[pl.ds(base, BPW)], u2_v)
    pltpu.sync_copy(off_h, off_v)
    off_b = off_v[...]
    iota = lax.iota(_i32, 16)

    cdf_last = _bcast(N - 1, cdf_v)
    for h in range(2):
        t1 = u1_v[pl.ds(h * 16, 16)] * cdf_last
        cnt = _search(lambda m: plsc.load_gather(cdf_v, [m]), t1, N, 11)
        tiv = jnp.minimum(cnt, jnp.full((16,), N - 1, _i32))
        ti_v[pl.ds(h * 16, 16)] = tiv
        tib_v[pl.ds(h * 16, 16)] = tiv * jnp.full((16,), T, _i32)

    field_in = (act_h, rew_h, don_h, val_h, ret_h, wgt_h)
    field_out = (act_o, rew_o, don_o, val_o, ret_o, wgt_o)
    row_cps = []
    for f in range(6):
        row_cps.append(pltpu.async_copy(field_in[f].at[ti_v], rows_v.at[f], sem_row))

    pltpu.async_copy(tcdf_h.at[ti_v], tcdf_v, sem_t).wait()
    st_h = []
    for h in range(2):
        rows_idx = iota + h * 16
        c_last = plsc.load_gather(tcdf_v, [rows_idx, jnp.full((16,), VR - 1, _i32)])
        t2 = u2_v[pl.ds(h * 16, 16)] * c_last
        cnt = _search(lambda m: plsc.load_gather(tcdf_v, [rows_idx, m]), t2, VR, 9)
        st = jnp.minimum(cnt, jnp.full((16,), VR - 1, _i32))
        st_h.append(st)
        st_v[pl.ds(h * 16, 16)] = st

    def build(c, _):
        c2 = c * 2
        ti0 = _bcast(c2, tib_v)
        ti1 = _bcast(c2 + 1, tib_v)
        s0 = _bcast(c2, st_v)
        s1 = _bcast(c2 + 1, st_v)
        for k in range(7):
            j = iota + k * 16
            je = jnp.minimum(j, jnp.full((16,), 99, _i32))
            sl = je >= jnp.full((16,), S, _i32)
            s = je - jnp.where(sl, jnp.full((16,), S, _i32), jnp.zeros((16,), _i32))
            pos = jnp.where(sl, s1, s0) + s + off_b
            pos = jnp.clip(pos, jnp.zeros((16,), _i32), jnp.full((16,), T - 1, _i32))
            idx_v[c, pl.ds(k * 16, 16)] = jnp.where(sl, ti1, ti0) + pos
        return 0

    lax.fori_loop(0, NCH, build, 0)

    ap_cps = []
    for c in range(NCH):
        ap_cps.append(pltpu.async_copy(ap2d.at[idx_v.at[c]],
                                       ap_b.at[pl.ds(c * CW, CW)], sem_ap))
    obs_bufs = (obs_b0, obs_b1)
    obs_sems = (sem_o0, sem_o1)
    obs_cps = {}
    for c in range(2):
        obs_cps[c] = pltpu.async_copy(obs2d.at[idx_v.at[c]], obs_bufs[c], obs_sems[c])

    for f in range(6):
        row_cps[f].wait()
    for f in range(6):

        def extract(s, _):
            sv = jnp.full((16,), s, _i32)
            for h in range(2):
                rows_idx = iota + h * 16
                pos = st_h[h] + sv + off_b
                pos = jnp.clip(pos, jnp.zeros((16,), _i32),
                               jnp.full((16,), T - 1, _i32))
                vals = plsc.load_gather(rows_v, [jnp.full((16,), f, _i32),
                                                 rows_idx, pos])
                plsc.store_scatter(sout_v, [rows_idx, sv], vals)
            return 0

        lax.fori_loop(0, S, extract, 0)
        pltpu.sync_copy(sout_v, field_out[f].at[pl.ds(base, BPW)])

    for c in range(NCH):
        obs_cps[c].wait()
        bb = base + 2 * c
        pltpu.sync_copy(obs_bufs[c % 2].at[pl.ds(0, S)], obs_o.at[:, bb])
        pltpu.sync_copy(obs_bufs[c % 2].at[pl.ds(S, S)], obs_o.at[:, bb + 1])
        if c + 2 < NCH:
            obs_cps[c + 2] = pltpu.async_copy(obs2d.at[idx_v.at[c + 2]],
                                              obs_bufs[c % 2], obs_sems[c % 2])

    for c in range(NCH):
        ap_cps[c].wait()
    for c in range(NCH):
        pltpu.sync_copy(ap_b.at[pl.ds(c * CW, 100)],
                        ap_o.at[pl.ds(base50 + c * 100, 100)])


@jax.jit
def _sample_gather(cdf, tcdf, u1, u2, off, obs2d, ap2d, act, rew, don, val, ret, wgt):
    f32 = jnp.float32
    return pl.kernel(
        _body,
        out_type=(
            jax.ShapeDtypeStruct((S, B, D_OBS), f32),
            jax.ShapeDtypeStruct((B * S, N_ACT), f32),
        ) + tuple(jax.ShapeDtypeStruct((B, S), f32) for _ in range(6)),
        mesh=plsc.VectorSubcoreMesh(core_axis_name="c", subcore_axis_name="s"),
        compiler_params=pltpu.CompilerParams(needs_layout_passes=False,
                                             use_tc_tiling_on_sc=False),
        scratch_types=[
            pltpu.VMEM((N,), f32),
            pltpu.VMEM((BPW, VRP), f32),
            pltpu.VMEM((BPW,), f32),
            pltpu.VMEM((BPW,), f32),
            pltpu.VMEM((BPW,), _i32),
            pltpu.VMEM((16,), _i32),
            pltpu.VMEM((BPW,), _i32),
            pltpu.VMEM((BPW,), _i32),
            pltpu.VMEM((NCH, CW), _i32),
            pltpu.VMEM((6, BPW, T), f32),
            pltpu.VMEM((BPW, S), f32),
            pltpu.VMEM((CW, D_OBS), f32),
            pltpu.VMEM((CW, D_OBS), f32),
            pltpu.VMEM((NCH * CW, N_ACT), f32),
            pltpu.SemaphoreType.DMA,
            pltpu.SemaphoreType.DMA,
            pltpu.SemaphoreType.DMA,
            pltpu.SemaphoreType.DMA,
            pltpu.SemaphoreType.DMA,
        ],
    )(cdf, tcdf, u1, u2, off, obs2d, ap2d, act, rew, don, val, ret, wgt)


def kernel(obs, action, reward, done, value, action_probs, returns, weight,
           weights, u1, u2, steps):
    p = weights / jnp.sum(weights)
    cdf = jnp.cumsum(p)
    tw = weight[:, :VR]
    tw_norm = tw / (jnp.sum(tw, axis=1, keepdims=True) + 1e-6)
    tcdf = jnp.cumsum(tw_norm, axis=1)
    tcdf_pad = jnp.concatenate([tcdf, jnp.zeros((B, VRP - VR), jnp.float32)], axis=1)
    off = jnp.full((16,), steps - S, _i32)
    outs = _sample_gather(
        cdf, tcdf_pad, u1, u2, off,
        obs.reshape(N * T, D_OBS), action_probs.reshape(N * T, N_ACT),
        action, reward, done, value, returns, weight)
    obs_o, ap_o, act_o, rew_o, don_o, val_o, ret_o, wgt_o = outs
    return (jnp.swapaxes(obs_o, 0, 1), act_o, rew_o, don_o, val_o,
            ap_o.reshape(B, S, N_ACT), ret_o, wgt_o)


# speedup vs baseline: 1.2317x; 1.2317x over previous
import jax
import jax.numpy as jnp
from jax import lax
from jax.experimental import pallas as pl
from jax.experimental.pallas import tpu as pltpu
from jax.experimental.pallas import tpu_sc as plsc

N = 1024
T = 200
D_OBS = 128
N_ACT = 16
B = 1024
S = 50
VR = T - S + 1

NW = 32
BPW = B // NW
NCH = BPW // 2
CW = 112
CWB = 104
VRP = 152

_i32 = jnp.int32


def _bcast(v, ref):
    return plsc.load_gather(ref, [jnp.full((16,), v, _i32)])


def _search(load_mid, t, hi, steps):
    lo0 = jnp.zeros((16,), _i32)
    hi0 = jnp.full((16,), hi, _i32)

    def step(_, carry):
        lo, hi = carry
        mid = lax.shift_right_arithmetic(lo + hi, jnp.full((16,), 1, _i32))
        pred = load_mid(mid) <= t
        return (jnp.where(pred, mid + 1, lo), jnp.where(pred, hi, mid))

    lo, _ = lax.fori_loop(0, steps, step, (lo0, hi0))
    return lo


def _body(cdf_h, tcdf_h, u1_h, u2_h, off_h,
          obs2d, ap2d, act_h, rew_h, don_h, val_h, ret_h, wgt_h,
          obs_o, ap_o, act_o, rew_o, don_o, val_o, ret_o, wgt_o,
          cdf_v, tcdf_v, u1_v, u2_v, ti_v, off_v, st_v, tib_v, idx_v,
          rows_v, sout_v, obs_b0, obs_b1, ap_b, ap_st,
          sem_t, sem_row, sem_ap, sem_o0, sem_o1):
    wid = lax.axis_index("s") * 2 + lax.axis_index("c")
    base = wid * BPW
    base50 = base * S

    pltpu.sync_copy(cdf_h, cdf_v)
    pltpu.sync_copy(u1_h.at[pl.ds(base, BPW)], u1_v)
    pltpu.sync_copy(u2_h.at[pl.ds(base, BPW)], u2_v)
    pltpu.sync_copy(off_h, off_v)
    off_b = off_v[...]
    iota = lax.iota(_i32, 16)

    cdf_last = _bcast(N - 1, cdf_v)
    for h in range(2):
        t1 = u1_v[pl.ds(h * 16, 16)] * cdf_last
        cnt = _search(lambda m: plsc.load_gather(cdf_v, [m]), t1, N, 11)
        tiv = jnp.minimum(cnt, jnp.full((16,), N - 1, _i32))
        ti_v[pl.ds(h * 16, 16)] = tiv
        tib_v[pl.ds(h * 16, 16)] = tiv * jnp.full((16,), T, _i32)

    field_in = (act_h, rew_h, don_h, val_h, ret_h, wgt_h)
    field_out = (act_o, rew_o, don_o, val_o, ret_o, wgt_o)
    row_cps = []
    for f in range(6):
        row_cps.append(pltpu.async_copy(field_in[f].at[ti_v], rows_v.at[f], sem_row))

    pltpu.async_copy(tcdf_h.at[ti_v], tcdf_v, sem_t).wait()
    st_h = []
    for h in range(2):
        rows_idx = iota + h * 16
        c_last = plsc.load_gather(tcdf_v, [rows_idx, jnp.full((16,), VR - 1, _i32)])
        t2 = u2_v[pl.ds(h * 16, 16)] * c_last
        cnt = _search(lambda m: plsc.load_gather(tcdf_v, [rows_idx, m]), t2, VR, 9)
        st = jnp.minimum(cnt, jnp.full((16,), VR - 1, _i32))
        st_h.append(st)
        st_v[pl.ds(h * 16, 16)] = st

    def build(c, _):
        c2 = c * 2
        ti0 = _bcast(c2, tib_v)
        ti1 = _bcast(c2 + 1, tib_v)
        s0 = _bcast(c2, st_v)
        s1 = _bcast(c2 + 1, st_v)
        for k in range(7):
            j = iota + k * 16
            je = jnp.minimum(j, jnp.full((16,), 99, _i32))
            sl = je >= jnp.full((16,), S, _i32)
            s = je - jnp.where(sl, jnp.full((16,), S, _i32), jnp.zeros((16,), _i32))
            pos = jnp.where(sl, s1, s0) + s + off_b
            pos = jnp.clip(pos, jnp.zeros((16,), _i32), jnp.full((16,), T - 1, _i32))
            idx_v[c, pl.ds(k * 16, 16)] = jnp.where(sl, ti1, ti0) + pos
        return 0

    lax.fori_loop(0, NCH, build, 0)

    ap_cps = []
    for c in range(NCH):
        ap_cps.append(pltpu.async_copy(ap2d.at[idx_v.at[c, pl.ds(0, CWB)]],
                                       ap_b.at[pl.ds(c * CWB, CWB)], sem_ap))
    obs_bufs = (obs_b0, obs_b1)
    obs_sems = (sem_o0, sem_o1)
    obs_cps = {}
    for c in range(2):
        obs_cps[c] = pltpu.async_copy(obs2d.at[idx_v.at[c, pl.ds(0, CWB)]],
                                      obs_bufs[c], obs_sems[c])

    for f in range(6):
        row_cps[f].wait()
    for f in range(6):

        def extract(s, _):
            sv = jnp.full((16,), s, _i32)
            for h in range(2):
                rows_idx = iota + h * 16
                pos = st_h[h] + sv + off_b
                pos = jnp.clip(pos, jnp.zeros((16,), _i32),
                               jnp.full((16,), T - 1, _i32))
                vals = plsc.load_gather(rows_v, [jnp.full((16,), f, _i32),
                                                 rows_idx, pos])
                plsc.store_scatter(sout_v, [rows_idx, sv], vals)
            return 0

        lax.fori_loop(0, S, extract, 0)
        pltpu.sync_copy(sout_v, field_out[f].at[pl.ds(base, BPW)])

    for c in range(NCH):
        obs_cps[c].wait()
        bb = base + 2 * c
        pltpu.sync_copy(obs_bufs[c % 2].at[pl.ds(0, S)], obs_o.at[:, bb])
        pltpu.sync_copy(obs_bufs[c % 2].at[pl.ds(S, S)], obs_o.at[:, bb + 1])
        if c + 2 < NCH:
            obs_cps[c + 2] = pltpu.async_copy(obs2d.at[idx_v.at[c + 2, pl.ds(0, CWB)]],
                                              obs_bufs[c % 2], obs_sems[c % 2])

    for c in range(NCH):
        ap_cps[c].wait()
    a_hi = lax.shift_right_arithmetic(iota, jnp.full((16,), 3, _i32))
    a_lo = jnp.bitwise_and(iota, jnp.full((16,), 7, _i32))

    def ap_reorder(s, _):
        sv = jnp.full((16,), s, _i32)
        for bl in range(BPW):
            row = (bl // 2) * CWB + (bl % 2) * S
            vals = plsc.load_gather(ap_b, [jnp.full((16,), row, _i32) + sv, iota])
            plsc.store_scatter(ap_st, [sv, a_hi, a_lo, jnp.full((16,), bl, _i32)],
                               vals)
        return 0

    lax.fori_loop(0, S, ap_reorder, 0)
    b1c = lax.shift_right_arithmetic(base, 7)
    b2o = pl.multiple_of(jnp.bitwise_and(base, 127), 32)
    pltpu.sync_copy(ap_st, ap_o.at[:, :, b1c, :, pl.ds(b2o, BPW)])


@jax.jit
def _sample_gather(cdf, tcdf, u1, u2, off, obs2d, ap2d, act, rew, don, val, ret, wgt):
    f32 = jnp.float32
    return pl.kernel(
        _body,
        out_type=(
            jax.ShapeDtypeStruct((S, B, D_OBS), f32),
            jax.ShapeDtypeStruct((S, 2, 8, 8, 128), f32),
        ) + tuple(jax.ShapeDtypeStruct((B, S), f32) for _ in range(6)),
        mesh=plsc.VectorSubcoreMesh(core_axis_name="c", subcore_axis_name="s"),
        compiler_params=pltpu.CompilerParams(needs_layout_passes=False,
                                             use_tc_tiling_on_sc=False),
        scratch_types=[
            pltpu.VMEM((N,), f32),
            pltpu.VMEM((BPW, VRP), f32),
            pltpu.VMEM((BPW,), f32),
            pltpu.VMEM((BPW,), f32),
            pltpu.VMEM((BPW,), _i32),
            pltpu.VMEM((16,), _i32),
            pltpu.VMEM((BPW,), _i32),
            pltpu.VMEM((BPW,), _i32),
            pltpu.VMEM((NCH, CW), _i32),
            pltpu.VMEM((6, BPW, T), f32),
            pltpu.VMEM((BPW, S), f32),
            pltpu.VMEM((CWB, D_OBS), f32),
            pltpu.VMEM((CWB, D_OBS), f32),
            pltpu.VMEM((NCH * CWB, N_ACT), f32),
            pltpu.VMEM((S, 2, 8, BPW), f32),
            pltpu.SemaphoreType.DMA,
            pltpu.SemaphoreType.DMA,
            pltpu.SemaphoreType.DMA,
            pltpu.SemaphoreType.DMA,
            pltpu.SemaphoreType.DMA,
        ],
    )(cdf, tcdf, u1, u2, off, obs2d, ap2d, act, rew, don, val, ret, wgt)


def kernel(obs, action, reward, done, value, action_probs, returns, weight,
           weights, u1, u2, steps):
    p = weights / jnp.sum(weights)
    cdf = jnp.cumsum(p)
    tw = weight[:, :VR]
    tw_norm = tw / (jnp.sum(tw, axis=1, keepdims=True) + 1e-6)
    tcdf = jnp.cumsum(tw_norm, axis=1)
    tcdf_pad = jnp.concatenate([tcdf, jnp.zeros((B, VRP - VR), jnp.float32)], axis=1)
    off = jnp.full((16,), steps - S, _i32)
    outs = _sample_gather(
        cdf, tcdf_pad, u1, u2, off,
        obs.reshape(N * T, D_OBS), action_probs.reshape(N * T, N_ACT),
        action, reward, done, value, returns, weight)
    obs_o, ap_o, act_o, rew_o, don_o, val_o, ret_o, wgt_o = outs
    ap_out = jnp.transpose(ap_o, (2, 4, 0, 1, 3)).reshape(B, S, N_ACT)
    return (jnp.swapaxes(obs_o, 0, 1), act_o, rew_o, don_o, val_o,
            ap_out, ret_o, wgt_o)
